# baseline (speedup 1.0000x reference)
import functools

import jax
import jax.numpy as jnp
from jax import lax
from jax.experimental import pallas as pl
from jax.experimental.pallas import tpu as pltpu
from jax.experimental.pallas import tpu_sc as plsc

N = 10000
E = 320000
D = 128
HID = 64

NC = 2
NS = 16
LANES = 16
NW = NC * NS

CH = 128
NCHUNK = 80
EPW = CH * NCHUNK
EPAD = EPW * NW
TOTCH = EPAD // CH
CS0 = 45
CS1 = 115
CSF = max(CS0, CS1)
NPAD = 10240
RPW = NPAD // NS
ZR = 32
NBUF = 5

_mesh = plsc.VectorSubcoreMesh(core_axis_name="c", subcore_axis_name="s")


def _sc_deg_body(dst_hbm, ew_hbm, out_hbm, dstv, eww, zbuf, acc, sem):
    c = lax.axis_index("c")
    s = lax.axis_index("s")
    w = c * NS + s

    pltpu.sync_copy(dst_hbm.at[w], dstv)
    pltpu.sync_copy(ew_hbm.at[w], eww)

    def zf(i, _):
        zbuf[pl.ds(i * LANES, LANES)] = jnp.zeros((LANES,), jnp.float32)
        return 0
    lax.fori_loop(0, RPW // LANES, zf, 0)
    pltpu.sync_copy(zbuf, acc.at[pl.ds(s * RPW, RPW)])
    plsc.subcore_barrier()

    def fire(g, _):
        pltpu.async_copy(eww.at[g], acc.at[dstv.at[g]], sem, add=True)
        return 0
    lax.fori_loop(0, NCHUNK, fire, 0)

    def drain(g, _):
        pltpu.make_async_copy(eww.at[g], acc.at[dstv.at[g]], sem).wait()
        return 0
    lax.fori_loop(0, NCHUNK, drain, 0)

    plsc.subcore_barrier()
    pltpu.sync_copy(acc.at[pl.ds(s * RPW, RPW)], out_hbm.at[c, pl.ds(s * RPW, RPW)])


@functools.partial(
    pl.kernel,
    out_type=jax.ShapeDtypeStruct((NC, NPAD), jnp.float32),
    mesh=_mesh,
    scratch_types=[
        pltpu.VMEM((NCHUNK, CH), jnp.int32),
        pltpu.VMEM((NCHUNK, CH), jnp.float32),
        pltpu.VMEM((RPW,), jnp.float32),
        pltpu.VMEM_SHARED((NPAD,), jnp.float32),
        pltpu.SemaphoreType.DMA,
    ],
    compiler_params=pltpu.CompilerParams(use_tc_tiling_on_sc=False),
)
def _sc_deg(dst_hbm, ew_hbm, out_hbm, dstv, eww, zbuf, acc, sem):
    _sc_deg_body(dst_hbm, ew_hbm, out_hbm, dstv, eww, zbuf, acc, sem)


def _sc_agg_body(y_hbm, src_hbm, dst_hbm, ew_hbm, out_hbm,
                 srcv, dstv, eww, rows, zbuf, acc, gsem, ssem):
    c = lax.axis_index("c")
    s = lax.axis_index("s")

    cnt = jnp.where(c == 0, CS0, CS1)
    start = c * (NS * CS0) + s * cnt
    pltpu.sync_copy(src_hbm.at[pl.ds(start, CSF)], srcv)
    pltpu.sync_copy(dst_hbm.at[pl.ds(start, CSF)], dstv)
    pltpu.sync_copy(ew_hbm.at[pl.ds(start, CSF)], eww)

    def zf(i, _):
        for q in range(HID // LANES):
            zbuf[i, pl.ds(q * LANES, LANES)] = jnp.zeros((LANES,), jnp.float32)
        return 0
    lax.fori_loop(0, ZR, zf, 0)
    for r in range(RPW // ZR):
        pltpu.sync_copy(zbuf, acc.at[pl.ds(s * RPW + r * ZR, ZR)])
    plsc.subcore_barrier()

    def g_start(g, b):
        pltpu.async_copy(y_hbm.at[srcv.at[g]], rows.at[b], gsem)

    def g_wait(g, b):
        pltpu.make_async_copy(y_hbm.at[srcv.at[g]], rows.at[b], gsem).wait()

    def s_start(g, b):
        pltpu.async_copy(rows.at[b], acc.at[dstv.at[g]], ssem, add=True)

    def s_wait(g, b):
        pltpu.make_async_copy(rows.at[b], acc.at[dstv.at[g]], ssem).wait()

    def scale(g, b):
        @plsc.parallel_loop(0, CH // LANES, unroll=2)
        def sc16(j):
            ev = eww[g, pl.ds(j * LANES, LANES)]
            for k in range(LANES):
                nv = jnp.full((LANES,), ev[k], jnp.float32)
                e = j * LANES + k
                vals = [rows[b, e, pl.ds(q * LANES, LANES)] * nv
                        for q in range(HID // LANES)]
                for q in range(HID // LANES):
                    rows[b, e, pl.ds(q * LANES, LANES)] = vals[q]

    for b in range(NBUF):
        g_start(b, b)

    def outer(t, _):
        base = t * NBUF
        for b in range(NBUF):
            g = base + b
            g_wait(g, b)
            scale(g, b)
            s_start(g, b)
        for b in range(NBUF):
            g = base + b
            s_wait(g, b)

            @pl.when(g + NBUF < cnt)
            def _():
                g_start(g + NBUF, b)
        return 0
    lax.fori_loop(0, cnt // NBUF, outer, 0)

    plsc.subcore_barrier()
    pltpu.sync_copy(acc.at[pl.ds(s * RPW, RPW)],
                    out_hbm.at[c, pl.ds(s * RPW, RPW)])


@functools.partial(
    pl.kernel,
    out_type=jax.ShapeDtypeStruct((NC, NPAD, HID), jnp.float32),
    mesh=_mesh,
    scratch_types=[
        pltpu.VMEM((CSF, CH), jnp.int32),
        pltpu.VMEM((CSF, CH), jnp.int32),
        pltpu.VMEM((CSF, CH), jnp.float32),
        pltpu.VMEM((NBUF, CH, HID), jnp.float32),
        pltpu.VMEM((ZR, HID), jnp.float32),
        pltpu.VMEM_SHARED((NPAD, HID), jnp.float32),
        pltpu.SemaphoreType.DMA,
        pltpu.SemaphoreType.DMA,
    ],
    compiler_params=pltpu.CompilerParams(use_tc_tiling_on_sc=False),
)
def _sc_agg(y_hbm, src_hbm, dst_hbm, ew_hbm, out_hbm,
            srcv, dstv, eww, rows, zbuf, acc, gsem, ssem):
    _sc_agg_body(y_hbm, src_hbm, dst_hbm, ew_hbm, out_hbm,
                 srcv, dstv, eww, rows, zbuf, acc, gsem, ssem)


BR = 2000


def _tc_a_kern(x_ref, w_ref, dis_ref, xw_ref, y_ref):
    xw = jnp.dot(x_ref[...], w_ref[...], preferred_element_type=jnp.float32)
    xw_ref[...] = xw
    y_ref[...] = xw * dis_ref[...]


def _tc_a(x, W1, disv):
    grid = (N // BR,)
    return pl.pallas_call(
        _tc_a_kern,
        grid=grid,
        in_specs=[
            pl.BlockSpec((BR, D), lambda i: (i, 0)),
            pl.BlockSpec((D, HID), lambda i: (0, 0)),
            pl.BlockSpec((BR, 1), lambda i: (i, 0)),
        ],
        out_specs=[
            pl.BlockSpec((BR, HID), lambda i: (i, 0)),
            pl.BlockSpec((BR, HID), lambda i: (i, 0)),
        ],
        out_shape=[
            jax.ShapeDtypeStruct((N, HID), jnp.float32),
            jax.ShapeDtypeStruct((N, HID), jnp.float32),
        ],
    )(x, W1, disv)


def _tc_b_kern(zp_ref, xw_ref, dis_ref, dinv_ref, b_ref, h_ref, y2_ref):
    z = zp_ref[0] + zp_ref[1]
    h = jnp.maximum(z * dis_ref[...] + xw_ref[...] * dinv_ref[...] + b_ref[...],
                    0.0)
    h_ref[...] = h
    y2_ref[...] = h * dis_ref[...]


def _tc_b(zp, xw1, disv, dinvv, b1):
    grid = (N // BR,)
    return pl.pallas_call(
        _tc_b_kern,
        grid=grid,
        in_specs=[
            pl.BlockSpec((NC, BR, HID), lambda i: (0, i, 0)),
            pl.BlockSpec((BR, HID), lambda i: (i, 0)),
            pl.BlockSpec((BR, 1), lambda i: (i, 0)),
            pl.BlockSpec((BR, 1), lambda i: (i, 0)),
            pl.BlockSpec((1, HID), lambda i: (0, 0)),
        ],
        out_specs=[
            pl.BlockSpec((BR, HID), lambda i: (i, 0)),
            pl.BlockSpec((BR, HID), lambda i: (i, 0)),
        ],
        out_shape=[
            jax.ShapeDtypeStruct((N, HID), jnp.float32),
            jax.ShapeDtypeStruct((N, HID), jnp.float32),
        ],
    )(zp, xw1, disv, dinvv, b1)


def _tc_c_kern(zp_ref, h_ref, dis_ref, dinv_ref, w_ref, b_ref, o_ref):
    g = (zp_ref[0] + zp_ref[1]) * dis_ref[...] + h_ref[...] * dinv_ref[...]
    o_ref[...] = (jnp.dot(g, w_ref[...], preferred_element_type=jnp.float32)
                  + b_ref[...])


def _tc_c(zp, h, disv, dinvv, W2, b2):
    grid = (N // BR,)
    return pl.pallas_call(
        _tc_c_kern,
        grid=grid,
        in_specs=[
            pl.BlockSpec((NC, BR, HID), lambda i: (0, i, 0)),
            pl.BlockSpec((BR, HID), lambda i: (i, 0)),
            pl.BlockSpec((BR, 1), lambda i: (i, 0)),
            pl.BlockSpec((BR, 1), lambda i: (i, 0)),
            pl.BlockSpec((HID, D), lambda i: (0, 0)),
            pl.BlockSpec((1, D), lambda i: (0, 0)),
        ],
        out_specs=pl.BlockSpec((BR, D), lambda i: (i, 0)),
        out_shape=jax.ShapeDtypeStruct((N, D), jnp.float32),
    )(zp, h, disv, dinvv, W2, b2)


def kernel(x, edge_index, edge_attr, W1, b1, W2, b2):
    src = edge_index[0].astype(jnp.int32)
    dst = edge_index[1].astype(jnp.int32)
    pad = EPAD - E
    src = jnp.concatenate([src, jnp.zeros((pad,), jnp.int32)])
    dst = jnp.concatenate([dst, jnp.zeros((pad,), jnp.int32)])
    ew = jnp.concatenate([edge_attr, jnp.zeros((pad,), jnp.float32)])
    srcc = src.reshape(TOTCH, CH)
    dstc = dst.reshape(TOTCH, CH)
    ewc = ew.reshape(TOTCH, CH)
    dst3 = dst.reshape(NW, NCHUNK, CH)
    ew3 = ew.reshape(NW, NCHUNK, CH)

    deg_parts = _sc_deg(dst3, ew3)
    deg = deg_parts[0, :N] + deg_parts[1, :N] + 1.0
    disv = lax.rsqrt(deg)[:, None]
    dinvv = (1.0 / deg)[:, None]

    xw1, y1 = _tc_a(x, W1, disv)
    z1p = _sc_agg(y1, srcc, dstc, ewc)
    h, y2 = _tc_b(z1p[:, :N], xw1, disv, dinvv, b1[None, :])
    z2p = _sc_agg(y2, srcc, dstc, ewc)
    out = _tc_c(z2p[:, :N], h, disv, dinvv, W2, b2[None, :])
    return out


# speedup vs baseline: 1.1158x; 1.1158x over previous
import functools

import jax
import jax.numpy as jnp
from jax import lax
from jax.experimental import pallas as pl
from jax.experimental.pallas import tpu as pltpu
from jax.experimental.pallas import tpu_sc as plsc

N = 10000
E = 320000
D = 128
HID = 64

NC = 2
NS = 16
LANES = 16
NW = NC * NS

CH = 128
NCHUNK = 80
EPW = CH * NCHUNK
EPAD = EPW * NW
TOTCH = EPAD // CH
CS0 = 115
CS1 = 45
CSF = max(CS0, CS1)
NPAD = 10240
RPW = NPAD // NS
ZR = 32
NBUF = 5

_mesh = plsc.VectorSubcoreMesh(core_axis_name="c", subcore_axis_name="s")


def _sc_deg_body(dst_hbm, ew_hbm, out_hbm, dstv, eww, zbuf, acc, sem):
    c = lax.axis_index("c")
    s = lax.axis_index("s")
    w = c * NS + s

    pltpu.sync_copy(dst_hbm.at[w], dstv)
    pltpu.sync_copy(ew_hbm.at[w], eww)

    def zf(i, _):
        zbuf[pl.ds(i * LANES, LANES)] = jnp.zeros((LANES,), jnp.float32)
        return 0
    lax.fori_loop(0, RPW // LANES, zf, 0)
    pltpu.sync_copy(zbuf, acc.at[pl.ds(s * RPW, RPW)])
    plsc.subcore_barrier()

    def fire(g, _):
        pltpu.async_copy(eww.at[g], acc.at[dstv.at[g]], sem, add=True)
        return 0
    lax.fori_loop(0, NCHUNK, fire, 0)

    def drain(g, _):
        pltpu.make_async_copy(eww.at[g], acc.at[dstv.at[g]], sem).wait()
        return 0
    lax.fori_loop(0, NCHUNK, drain, 0)

    plsc.subcore_barrier()
    pltpu.sync_copy(acc.at[pl.ds(s * RPW, RPW)], out_hbm.at[c, pl.ds(s * RPW, RPW)])


@functools.partial(
    pl.kernel,
    out_type=jax.ShapeDtypeStruct((NC, NPAD), jnp.float32),
    mesh=_mesh,
    scratch_types=[
        pltpu.VMEM((NCHUNK, CH), jnp.int32),
        pltpu.VMEM((NCHUNK, CH), jnp.float32),
        pltpu.VMEM((RPW,), jnp.float32),
        pltpu.VMEM_SHARED((NPAD,), jnp.float32),
        pltpu.SemaphoreType.DMA,
    ],
    compiler_params=pltpu.CompilerParams(use_tc_tiling_on_sc=False),
)
def _sc_deg(dst_hbm, ew_hbm, out_hbm, dstv, eww, zbuf, acc, sem):
    _sc_deg_body(dst_hbm, ew_hbm, out_hbm, dstv, eww, zbuf, acc, sem)


def _sc_agg_body(y_hbm, src_hbm, dst_hbm, ew_hbm, out_hbm,
                 srcv, dstv, eww, rows, zbuf, acc, gsem, ssem):
    c = lax.axis_index("c")
    s = lax.axis_index("s")

    cnt = jnp.where(c == 0, CS0, CS1)
    start = c * (NS * CS0) + s * cnt
    pltpu.sync_copy(src_hbm.at[pl.ds(start, CSF)], srcv)
    pltpu.sync_copy(dst_hbm.at[pl.ds(start, CSF)], dstv)
    pltpu.sync_copy(ew_hbm.at[pl.ds(start, CSF)], eww)

    def zf(i, _):
        for q in range(HID // LANES):
            zbuf[i, pl.ds(q * LANES, LANES)] = jnp.zeros((LANES,), jnp.float32)
        return 0
    lax.fori_loop(0, ZR, zf, 0)
    for r in range(RPW // ZR):
        pltpu.sync_copy(zbuf, acc.at[pl.ds(s * RPW + r * ZR, ZR)])
    plsc.subcore_barrier()

    def g_start(g, b):
        pltpu.async_copy(y_hbm.at[srcv.at[g]], rows.at[b], gsem)

    def g_wait(g, b):
        pltpu.make_async_copy(y_hbm.at[srcv.at[g]], rows.at[b], gsem).wait()

    def s_start(g, b):
        pltpu.async_copy(rows.at[b], acc.at[dstv.at[g]], ssem, add=True)

    def s_wait(g, b):
        pltpu.make_async_copy(rows.at[b], acc.at[dstv.at[g]], ssem).wait()

    def scale(g, b):
        @plsc.parallel_loop(0, CH // LANES, unroll=2)
        def sc16(j):
            ev = eww[g, pl.ds(j * LANES, LANES)]
            for k in range(LANES):
                nv = jnp.full((LANES,), ev[k], jnp.float32)
                e = j * LANES + k
                vals = [rows[b, e, pl.ds(q * LANES, LANES)] * nv
                        for q in range(HID // LANES)]
                for q in range(HID // LANES):
                    rows[b, e, pl.ds(q * LANES, LANES)] = vals[q]

    for b in range(NBUF):
        g_start(b, b)

    def outer(t, _):
        base = t * NBUF
        for b in range(NBUF):
            g = base + b
            g_wait(g, b)
            scale(g, b)
            s_start(g, b)
        for b in range(NBUF):
            g = base + b
            s_wait(g, b)

            @pl.when(g + NBUF < cnt)
            def _():
                g_start(g + NBUF, b)
        return 0
    lax.fori_loop(0, cnt // NBUF, outer, 0)

    plsc.subcore_barrier()
    pltpu.sync_copy(acc.at[pl.ds(s * RPW, RPW)],
                    out_hbm.at[c, pl.ds(s * RPW, RPW)])


@functools.partial(
    pl.kernel,
    out_type=jax.ShapeDtypeStruct((NC, NPAD, HID), jnp.float32),
    mesh=_mesh,
    scratch_types=[
        pltpu.VMEM((CSF, CH), jnp.int32),
        pltpu.VMEM((CSF, CH), jnp.int32),
        pltpu.VMEM((CSF, CH), jnp.float32),
        pltpu.VMEM((NBUF, CH, HID), jnp.float32),
        pltpu.VMEM((ZR, HID), jnp.float32),
        pltpu.VMEM_SHARED((NPAD, HID), jnp.float32),
        pltpu.SemaphoreType.DMA,
        pltpu.SemaphoreType.DMA,
    ],
    compiler_params=pltpu.CompilerParams(use_tc_tiling_on_sc=False),
)
def _sc_agg(y_hbm, src_hbm, dst_hbm, ew_hbm, out_hbm,
            srcv, dstv, eww, rows, zbuf, acc, gsem, ssem):
    _sc_agg_body(y_hbm, src_hbm, dst_hbm, ew_hbm, out_hbm,
                 srcv, dstv, eww, rows, zbuf, acc, gsem, ssem)


BR = 2000


def _tc_a_kern(x_ref, w_ref, dis_ref, xw_ref, y_ref):
    xw = jnp.dot(x_ref[...], w_ref[...], preferred_element_type=jnp.float32)
    xw_ref[...] = xw
    y_ref[...] = xw * dis_ref[...]


def _tc_a(x, W1, disv):
    grid = (N // BR,)
    return pl.pallas_call(
        _tc_a_kern,
        grid=grid,
        in_specs=[
            pl.BlockSpec((BR, D), lambda i: (i, 0)),
            pl.BlockSpec((D, HID), lambda i: (0, 0)),
            pl.BlockSpec((BR, 1), lambda i: (i, 0)),
        ],
        out_specs=[
            pl.BlockSpec((BR, HID), lambda i: (i, 0)),
            pl.BlockSpec((BR, HID), lambda i: (i, 0)),
        ],
        out_shape=[
            jax.ShapeDtypeStruct((N, HID), jnp.float32),
            jax.ShapeDtypeStruct((N, HID), jnp.float32),
        ],
    )(x, W1, disv)


def _tc_b_kern(zp_ref, xw_ref, dis_ref, dinv_ref, b_ref, h_ref, y2_ref):
    z = zp_ref[0] + zp_ref[1]
    h = jnp.maximum(z * dis_ref[...] + xw_ref[...] * dinv_ref[...] + b_ref[...],
                    0.0)
    h_ref[...] = h
    y2_ref[...] = h * dis_ref[...]


def _tc_b(zp, xw1, disv, dinvv, b1):
    grid = (N // BR,)
    return pl.pallas_call(
        _tc_b_kern,
        grid=grid,
        in_specs=[
            pl.BlockSpec((NC, BR, HID), lambda i: (0, i, 0)),
            pl.BlockSpec((BR, HID), lambda i: (i, 0)),
            pl.BlockSpec((BR, 1), lambda i: (i, 0)),
            pl.BlockSpec((BR, 1), lambda i: (i, 0)),
            pl.BlockSpec((1, HID), lambda i: (0, 0)),
        ],
        out_specs=[
            pl.BlockSpec((BR, HID), lambda i: (i, 0)),
            pl.BlockSpec((BR, HID), lambda i: (i, 0)),
        ],
        out_shape=[
            jax.ShapeDtypeStruct((N, HID), jnp.float32),
            jax.ShapeDtypeStruct((N, HID), jnp.float32),
        ],
    )(zp, xw1, disv, dinvv, b1)


def _tc_c_kern(zp_ref, h_ref, dis_ref, dinv_ref, w_ref, b_ref, o_ref):
    g = (zp_ref[0] + zp_ref[1]) * dis_ref[...] + h_ref[...] * dinv_ref[...]
    o_ref[...] = (jnp.dot(g, w_ref[...], preferred_element_type=jnp.float32)
                  + b_ref[...])


def _tc_c(zp, h, disv, dinvv, W2, b2):
    grid = (N // BR,)
    return pl.pallas_call(
        _tc_c_kern,
        grid=grid,
        in_specs=[
            pl.BlockSpec((NC, BR, HID), lambda i: (0, i, 0)),
            pl.BlockSpec((BR, HID), lambda i: (i, 0)),
            pl.BlockSpec((BR, 1), lambda i: (i, 0)),
            pl.BlockSpec((BR, 1), lambda i: (i, 0)),
            pl.BlockSpec((HID, D), lambda i: (0, 0)),
            pl.BlockSpec((1, D), lambda i: (0, 0)),
        ],
        out_specs=pl.BlockSpec((BR, D), lambda i: (i, 0)),
        out_shape=jax.ShapeDtypeStruct((N, D), jnp.float32),
    )(zp, h, disv, dinvv, W2, b2)


def kernel(x, edge_index, edge_attr, W1, b1, W2, b2):
    src = edge_index[0].astype(jnp.int32)
    dst = edge_index[1].astype(jnp.int32)
    pad = EPAD - E
    src = jnp.concatenate([src, jnp.zeros((pad,), jnp.int32)])
    dst = jnp.concatenate([dst, jnp.zeros((pad,), jnp.int32)])
    ew = jnp.concatenate([edge_attr, jnp.zeros((pad,), jnp.float32)])
    srcc = src.reshape(TOTCH, CH)
    dstc = dst.reshape(TOTCH, CH)
    ewc = ew.reshape(TOTCH, CH)
    dst3 = dst.reshape(NW, NCHUNK, CH)
    ew3 = ew.reshape(NW, NCHUNK, CH)

    deg_parts = _sc_deg(dst3, ew3)
    deg = deg_parts[0, :N] + deg_parts[1, :N] + 1.0
    disv = lax.rsqrt(deg)[:, None]
    dinvv = (1.0 / deg)[:, None]

    xw1, y1 = _tc_a(x, W1, disv)
    z1p = _sc_agg(y1, srcc, dstc, ewc)
    h, y2 = _tc_b(z1p[:, :N], xw1, disv, dinvv, b1[None, :])
    z2p = _sc_agg(y2, srcc, dstc, ewc)
    out = _tc_c(z2p[:, :N], h, disv, dinvv, W2, b2[None, :])
    return out


# speedup vs baseline: 1.1176x; 1.0016x over previous
import functools

import jax
import jax.numpy as jnp
from jax import lax
from jax.experimental import pallas as pl
from jax.experimental.pallas import tpu as pltpu
from jax.experimental.pallas import tpu_sc as plsc

N = 10000
E = 320000
D = 128
HID = 64

NC = 2
NS = 16
LANES = 16
NW = NC * NS

CH = 128
NCHUNK = 80
EPW = CH * NCHUNK
EPAD = EPW * NW
TOTCH = EPAD // CH
CS0 = 115
CS1 = 45
CSF = max(CS0, CS1)
NPAD = 10240
RPW = NPAD // NS
ZR = 32
NBUF = 5

_mesh = plsc.VectorSubcoreMesh(core_axis_name="c", subcore_axis_name="s")


def _sc_deg_body(dst_hbm, ew_hbm, out_hbm, dstv, eww, zbuf, acc, sem):
    c = lax.axis_index("c")
    s = lax.axis_index("s")
    w = c * NS + s

    pltpu.sync_copy(dst_hbm.at[w], dstv)
    pltpu.sync_copy(ew_hbm.at[w], eww)

    def zf(i, _):
        zbuf[pl.ds(i * LANES, LANES)] = jnp.zeros((LANES,), jnp.float32)
        return 0
    lax.fori_loop(0, RPW // LANES, zf, 0)
    pltpu.sync_copy(zbuf, acc.at[pl.ds(s * RPW, RPW)])
    plsc.subcore_barrier()

    def fire(g, _):
        pltpu.async_copy(eww.at[g], acc.at[dstv.at[g]], sem, add=True)
        return 0
    lax.fori_loop(0, NCHUNK, fire, 0)

    def drain(g, _):
        pltpu.make_async_copy(eww.at[g], acc.at[dstv.at[g]], sem).wait()
        return 0
    lax.fori_loop(0, NCHUNK, drain, 0)

    plsc.subcore_barrier()
    pltpu.sync_copy(acc.at[pl.ds(s * RPW, RPW)], out_hbm.at[c, pl.ds(s * RPW, RPW)])


@functools.partial(
    pl.kernel,
    out_type=jax.ShapeDtypeStruct((NC, NPAD), jnp.float32),
    mesh=_mesh,
    scratch_types=[
        pltpu.VMEM((NCHUNK, CH), jnp.int32),
        pltpu.VMEM((NCHUNK, CH), jnp.float32),
        pltpu.VMEM((RPW,), jnp.float32),
        pltpu.VMEM_SHARED((NPAD,), jnp.float32),
        pltpu.SemaphoreType.DMA,
    ],
    compiler_params=pltpu.CompilerParams(use_tc_tiling_on_sc=False),
)
def _sc_deg(dst_hbm, ew_hbm, out_hbm, dstv, eww, zbuf, acc, sem):
    _sc_deg_body(dst_hbm, ew_hbm, out_hbm, dstv, eww, zbuf, acc, sem)


def _sc_agg_body(y_hbm, src_hbm, dst_hbm, ew_hbm, out_hbm,
                 srcv, dstv, eww, rows, zbuf, acc, gsem, ssem):
    c = lax.axis_index("c")
    s = lax.axis_index("s")

    cnt = jnp.where(c == 0, CS0, CS1)
    start = c * (NS * CS0) + s * cnt
    with jax.named_scope("stage_edges"):
        pltpu.sync_copy(src_hbm.at[pl.ds(start, CSF)], srcv)
        pltpu.sync_copy(dst_hbm.at[pl.ds(start, CSF)], dstv)
        pltpu.sync_copy(ew_hbm.at[pl.ds(start, CSF)], eww)

    with jax.named_scope("zero_acc"):
        def zf(i, _):
            for q in range(HID // LANES):
                zbuf[i, pl.ds(q * LANES, LANES)] = jnp.zeros((LANES,), jnp.float32)
            return 0
        lax.fori_loop(0, ZR, zf, 0)
        for r in range(RPW // ZR):
            pltpu.sync_copy(zbuf, acc.at[pl.ds(s * RPW + r * ZR, ZR)])
    with jax.named_scope("barrier0"):
        plsc.subcore_barrier()

    def g_start(g, b):
        pltpu.async_copy(y_hbm.at[srcv.at[g]], rows.at[b], gsem)

    def g_wait(g, b):
        pltpu.make_async_copy(y_hbm.at[srcv.at[g]], rows.at[b], gsem).wait()

    def s_start(g, b):
        pltpu.async_copy(rows.at[b], acc.at[dstv.at[g]], ssem, add=True)

    def s_wait(g, b):
        pltpu.make_async_copy(rows.at[b], acc.at[dstv.at[g]], ssem).wait()

    def scale(g, b):
        @plsc.parallel_loop(0, CH // LANES, unroll=2)
        def sc16(j):
            ev = eww[g, pl.ds(j * LANES, LANES)]
            for k in range(LANES):
                nv = jnp.full((LANES,), ev[k], jnp.float32)
                e = j * LANES + k
                vals = [rows[b, e, pl.ds(q * LANES, LANES)] * nv
                        for q in range(HID // LANES)]
                for q in range(HID // LANES):
                    rows[b, e, pl.ds(q * LANES, LANES)] = vals[q]

    with jax.named_scope("prime"):
        for b in range(NBUF):
            g_start(b, b)

    def outer(t, _):
        base = t * NBUF
        for b in range(NBUF):
            g = base + b
            g_wait(g, b)
            scale(g, b)
            s_start(g, b)
        for b in range(NBUF):
            g = base + b
            s_wait(g, b)

            @pl.when(g + NBUF < cnt)
            def _():
                g_start(g + NBUF, b)
        return 0
    with jax.named_scope("mainloop"):
        lax.fori_loop(0, cnt // NBUF, outer, 0)

    with jax.named_scope("barrier1"):
        plsc.subcore_barrier()
    with jax.named_scope("writeout"):
        pltpu.sync_copy(acc.at[pl.ds(s * RPW, RPW)],
                        out_hbm.at[c, pl.ds(s * RPW, RPW)])


@functools.partial(
    pl.kernel,
    out_type=jax.ShapeDtypeStruct((NC, NPAD, HID), jnp.float32),
    mesh=_mesh,
    scratch_types=[
        pltpu.VMEM((CSF, CH), jnp.int32),
        pltpu.VMEM((CSF, CH), jnp.int32),
        pltpu.VMEM((CSF, CH), jnp.float32),
        pltpu.VMEM((NBUF, CH, HID), jnp.float32),
        pltpu.VMEM((ZR, HID), jnp.float32),
        pltpu.VMEM_SHARED((NPAD, HID), jnp.float32),
        pltpu.SemaphoreType.DMA,
        pltpu.SemaphoreType.DMA,
    ],
    compiler_params=pltpu.CompilerParams(use_tc_tiling_on_sc=False),
)
def _sc_agg(y_hbm, src_hbm, dst_hbm, ew_hbm, out_hbm,
            srcv, dstv, eww, rows, zbuf, acc, gsem, ssem):
    _sc_agg_body(y_hbm, src_hbm, dst_hbm, ew_hbm, out_hbm,
                 srcv, dstv, eww, rows, zbuf, acc, gsem, ssem)


BR = 2000


def _tc_a_kern(x_ref, w_ref, dis_ref, xw_ref, y_ref):
    xw = jnp.dot(x_ref[...], w_ref[...], preferred_element_type=jnp.float32)
    xw_ref[...] = xw
    y_ref[...] = xw * dis_ref[...]


def _tc_a(x, W1, disv):
    grid = (N // BR,)
    return pl.pallas_call(
        _tc_a_kern,
        grid=grid,
        in_specs=[
            pl.BlockSpec((BR, D), lambda i: (i, 0)),
            pl.BlockSpec((D, HID), lambda i: (0, 0)),
            pl.BlockSpec((BR, 1), lambda i: (i, 0)),
        ],
        out_specs=[
            pl.BlockSpec((BR, HID), lambda i: (i, 0)),
            pl.BlockSpec((BR, HID), lambda i: (i, 0)),
        ],
        out_shape=[
            jax.ShapeDtypeStruct((N, HID), jnp.float32),
            jax.ShapeDtypeStruct((N, HID), jnp.float32),
        ],
    )(x, W1, disv)


def _tc_b_kern(zp_ref, xw_ref, dis_ref, dinv_ref, b_ref, h_ref, y2_ref):
    z = zp_ref[0] + zp_ref[1]
    h = jnp.maximum(z * dis_ref[...] + xw_ref[...] * dinv_ref[...] + b_ref[...],
                    0.0)
    h_ref[...] = h
    y2_ref[...] = h * dis_ref[...]


def _tc_b(zp, xw1, disv, dinvv, b1):
    grid = (N // BR,)
    return pl.pallas_call(
        _tc_b_kern,
        grid=grid,
        in_specs=[
            pl.BlockSpec((NC, BR, HID), lambda i: (0, i, 0)),
            pl.BlockSpec((BR, HID), lambda i: (i, 0)),
            pl.BlockSpec((BR, 1), lambda i: (i, 0)),
            pl.BlockSpec((BR, 1), lambda i: (i, 0)),
            pl.BlockSpec((1, HID), lambda i: (0, 0)),
        ],
        out_specs=[
            pl.BlockSpec((BR, HID), lambda i: (i, 0)),
            pl.BlockSpec((BR, HID), lambda i: (i, 0)),
        ],
        out_shape=[
            jax.ShapeDtypeStruct((N, HID), jnp.float32),
            jax.ShapeDtypeStruct((N, HID), jnp.float32),
        ],
    )(zp, xw1, disv, dinvv, b1)


def _tc_c_kern(zp_ref, h_ref, dis_ref, dinv_ref, w_ref, b_ref, o_ref):
    g = (zp_ref[0] + zp_ref[1]) * dis_ref[...] + h_ref[...] * dinv_ref[...]
    o_ref[...] = (jnp.dot(g, w_ref[...], preferred_element_type=jnp.float32)
                  + b_ref[...])


def _tc_c(zp, h, disv, dinvv, W2, b2):
    grid = (N // BR,)
    return pl.pallas_call(
        _tc_c_kern,
        grid=grid,
        in_specs=[
            pl.BlockSpec((NC, BR, HID), lambda i: (0, i, 0)),
            pl.BlockSpec((BR, HID), lambda i: (i, 0)),
            pl.BlockSpec((BR, 1), lambda i: (i, 0)),
            pl.BlockSpec((BR, 1), lambda i: (i, 0)),
            pl.BlockSpec((HID, D), lambda i: (0, 0)),
            pl.BlockSpec((1, D), lambda i: (0, 0)),
        ],
        out_specs=pl.BlockSpec((BR, D), lambda i: (i, 0)),
        out_shape=jax.ShapeDtypeStruct((N, D), jnp.float32),
    )(zp, h, disv, dinvv, W2, b2)


def kernel(x, edge_index, edge_attr, W1, b1, W2, b2):
    src = edge_index[0].astype(jnp.int32)
    dst = edge_index[1].astype(jnp.int32)
    pad = EPAD - E
    src = jnp.concatenate([src, jnp.zeros((pad,), jnp.int32)])
    dst = jnp.concatenate([dst, jnp.zeros((pad,), jnp.int32)])
    ew = jnp.concatenate([edge_attr, jnp.zeros((pad,), jnp.float32)])
    srcc = src.reshape(TOTCH, CH)
    dstc = dst.reshape(TOTCH, CH)
    ewc = ew.reshape(TOTCH, CH)
    dst3 = dst.reshape(NW, NCHUNK, CH)
    ew3 = ew.reshape(NW, NCHUNK, CH)

    deg_parts = _sc_deg(dst3, ew3)
    deg = deg_parts[0, :N] + deg_parts[1, :N] + 1.0
    disv = lax.rsqrt(deg)[:, None]
    dinvv = (1.0 / deg)[:, None]

    xw1, y1 = _tc_a(x, W1, disv)
    z1p = _sc_agg(y1, srcc, dstc, ewc)
    h, y2 = _tc_b(z1p[:, :N], xw1, disv, dinvv, b1[None, :])
    z2p = _sc_agg(y2, srcc, dstc, ewc)
    out = _tc_c(z2p[:, :N], h, disv, dinvv, W2, b2[None, :])
    return out


# speedup vs baseline: 2.6755x; 2.3941x over previous
import functools

import jax
import jax.numpy as jnp
from jax import lax
from jax.experimental import pallas as pl
from jax.experimental.pallas import tpu as pltpu
from jax.experimental.pallas import tpu_sc as plsc

N = 10000
E = 320000
D = 128
HID = 64

NC = 2
NS = 16
LANES = 16
NW = NC * NS

CH = 128
NCHUNK = 80
EPW = CH * NCHUNK
EPAD = EPW * NW
TOTCH = EPAD // CH
CS0 = 80
CS1 = 80
CSF = max(CS0, CS1)
NPAD = 10240
RPW = NPAD // NS
ZR = 32
NBUF = 5

_mesh = plsc.VectorSubcoreMesh(core_axis_name="c", subcore_axis_name="s")


def _sc_deg_body(dst_hbm, ew_hbm, out_hbm, dstv, eww, zbuf, acc, sem):
    c = lax.axis_index("c")
    s = lax.axis_index("s")
    w = c * NS + s

    pltpu.sync_copy(dst_hbm.at[w], dstv)
    pltpu.sync_copy(ew_hbm.at[w], eww)

    def zf(i, _):
        zbuf[pl.ds(i * LANES, LANES)] = jnp.zeros((LANES,), jnp.float32)
        return 0
    lax.fori_loop(0, RPW // LANES, zf, 0)
    pltpu.sync_copy(zbuf, acc.at[pl.ds(s * RPW, RPW)])
    plsc.subcore_barrier()

    def fire(g, _):
        pltpu.async_copy(eww.at[g], acc.at[dstv.at[g]], sem, add=True)
        return 0
    lax.fori_loop(0, NCHUNK, fire, 0)

    def drain(g, _):
        pltpu.make_async_copy(eww.at[g], acc.at[dstv.at[g]], sem).wait()
        return 0
    lax.fori_loop(0, NCHUNK, drain, 0)

    plsc.subcore_barrier()
    pltpu.sync_copy(acc.at[pl.ds(s * RPW, RPW)], out_hbm.at[c, pl.ds(s * RPW, RPW)])


@functools.partial(
    pl.kernel,
    out_type=jax.ShapeDtypeStruct((NC, NPAD), jnp.float32),
    mesh=_mesh,
    scratch_types=[
        pltpu.VMEM((NCHUNK, CH), jnp.int32),
        pltpu.VMEM((NCHUNK, CH), jnp.float32),
        pltpu.VMEM((RPW,), jnp.float32),
        pltpu.VMEM_SHARED((NPAD,), jnp.float32),
        pltpu.SemaphoreType.DMA,
    ],
    compiler_params=pltpu.CompilerParams(use_tc_tiling_on_sc=False),
)
def _sc_deg(dst_hbm, ew_hbm, out_hbm, dstv, eww, zbuf, acc, sem):
    _sc_deg_body(dst_hbm, ew_hbm, out_hbm, dstv, eww, zbuf, acc, sem)


def _sc_agg_body(y_hbm, src_hbm, dst_hbm, ew_hbm, out_hbm,
                 srcv, dstv, eww, rows, zbuf, acc, gsem, ssem):
    c = lax.axis_index("c")
    s = lax.axis_index("s")

    cnt = jnp.where(c == 0, CS0, CS1)
    start = c * (NS * CS0) + s * cnt
    pltpu.sync_copy(src_hbm.at[pl.ds(start, CSF)], srcv)
    pltpu.sync_copy(dst_hbm.at[pl.ds(start, CSF)], dstv)
    pltpu.sync_copy(ew_hbm.at[pl.ds(start, CSF)], eww)

    def zf(i, _):
        for q in range(HID // LANES):
            zbuf[i, pl.ds(q * LANES, LANES)] = jnp.zeros((LANES,), jnp.float32)
        return 0
    lax.fori_loop(0, ZR, zf, 0)
    for r in range(RPW // ZR):
        pltpu.sync_copy(zbuf, acc.at[pl.ds(s * RPW + r * ZR, ZR)])
    plsc.subcore_barrier()

    def g_start(g, b):
        pltpu.async_copy(y_hbm.at[srcv.at[g]], rows.at[b], gsem)

    def g_wait(g, b):
        pltpu.make_async_copy(y_hbm.at[srcv.at[g]], rows.at[b], gsem).wait()

    def s_start(g, b):
        pltpu.async_copy(rows.at[b], acc.at[dstv.at[g]], ssem, add=True)

    def s_wait(g, b):
        pltpu.make_async_copy(rows.at[b], acc.at[dstv.at[g]], ssem).wait()

    def scale(g, b):
        @plsc.parallel_loop(0, CH // LANES, unroll=2)
        def sc16(j):
            ev = eww[g, pl.ds(j * LANES, LANES)]
            for k in range(LANES):
                nv = jnp.full((LANES,), ev[k], jnp.float32)
                e = j * LANES + k
                vals = [rows[b, e, pl.ds(q * LANES, LANES)] * nv
                        for q in range(HID // LANES)]
                for q in range(HID // LANES):
                    rows[b, e, pl.ds(q * LANES, LANES)] = vals[q]

    for b in range(NBUF):
        g_start(b, b)

    def outer(t, _):
        base = t * NBUF
        for b in range(NBUF):
            g = base + b
            g_wait(g, b)
            scale(g, b)
            s_start(g, b)
        for b in range(NBUF):
            g = base + b
            s_wait(g, b)

            @pl.when(g + NBUF < cnt)
            def _():
                g_start(g + NBUF, b)
        return 0
    lax.fori_loop(0, cnt // NBUF, outer, 0)

    plsc.subcore_barrier()
    pltpu.sync_copy(acc.at[pl.ds(s * RPW, RPW)],
                    out_hbm.at[c, pl.ds(s * RPW, RPW)])


@functools.partial(
    pl.kernel,
    out_type=jax.ShapeDtypeStruct((NC, NPAD, HID), jnp.float32),
    mesh=_mesh,
    scratch_types=[
        pltpu.VMEM((CSF, CH), jnp.int32),
        pltpu.VMEM((CSF, CH), jnp.int32),
        pltpu.VMEM((CSF, CH), jnp.float32),
        pltpu.VMEM((NBUF, CH, HID), jnp.float32),
        pltpu.VMEM((ZR, HID), jnp.float32),
        pltpu.VMEM_SHARED((NPAD, HID), jnp.float32),
        pltpu.SemaphoreType.DMA,
        pltpu.SemaphoreType.DMA,
    ],
    compiler_params=pltpu.CompilerParams(use_tc_tiling_on_sc=False),
)
def _sc_agg(y_hbm, src_hbm, dst_hbm, ew_hbm, out_hbm,
            srcv, dstv, eww, rows, zbuf, acc, gsem, ssem):
    _sc_agg_body(y_hbm, src_hbm, dst_hbm, ew_hbm, out_hbm,
                 srcv, dstv, eww, rows, zbuf, acc, gsem, ssem)


BR = 2000


def _tc_a_kern(x_ref, w_ref, dis_ref, xw_ref, y_ref):
    xw = jnp.dot(x_ref[...], w_ref[...], preferred_element_type=jnp.float32)
    xw_ref[...] = xw
    y_ref[...] = xw * dis_ref[...]


def _tc_a(x, W1, disv):
    grid = (N // BR,)
    return pl.pallas_call(
        _tc_a_kern,
        grid=grid,
        in_specs=[
            pl.BlockSpec((BR, D), lambda i: (i, 0)),
            pl.BlockSpec((D, HID), lambda i: (0, 0)),
            pl.BlockSpec((BR, 1), lambda i: (i, 0)),
        ],
        out_specs=[
            pl.BlockSpec((BR, HID), lambda i: (i, 0)),
            pl.BlockSpec((BR, HID), lambda i: (i, 0)),
        ],
        out_shape=[
            jax.ShapeDtypeStruct((N, HID), jnp.float32),
            jax.ShapeDtypeStruct((N, HID), jnp.float32),
        ],
    )(x, W1, disv)


def _tc_b_kern(zp_ref, xw_ref, dis_ref, dinv_ref, b_ref, h_ref, y2_ref):
    z = zp_ref[0] + zp_ref[1]
    h = jnp.maximum(z * dis_ref[...] + xw_ref[...] * dinv_ref[...] + b_ref[...],
                    0.0)
    h_ref[...] = h
    y2_ref[...] = h * dis_ref[...]


def _tc_b(zp, xw1, disv, dinvv, b1):
    grid = (N // BR,)
    return pl.pallas_call(
        _tc_b_kern,
        grid=grid,
        in_specs=[
            pl.BlockSpec((NC, BR, HID), lambda i: (0, i, 0)),
            pl.BlockSpec((BR, HID), lambda i: (i, 0)),
            pl.BlockSpec((BR, 1), lambda i: (i, 0)),
            pl.BlockSpec((BR, 1), lambda i: (i, 0)),
            pl.BlockSpec((1, HID), lambda i: (0, 0)),
        ],
        out_specs=[
            pl.BlockSpec((BR, HID), lambda i: (i, 0)),
            pl.BlockSpec((BR, HID), lambda i: (i, 0)),
        ],
        out_shape=[
            jax.ShapeDtypeStruct((N, HID), jnp.float32),
            jax.ShapeDtypeStruct((N, HID), jnp.float32),
        ],
    )(zp, xw1, disv, dinvv, b1)


def _tc_c_kern(zp_ref, h_ref, dis_ref, dinv_ref, w_ref, b_ref, o_ref):
    g = (zp_ref[0] + zp_ref[1]) * dis_ref[...] + h_ref[...] * dinv_ref[...]
    o_ref[...] = (jnp.dot(g, w_ref[...], preferred_element_type=jnp.float32)
                  + b_ref[...])


def _tc_c(zp, h, disv, dinvv, W2, b2):
    grid = (N // BR,)
    return pl.pallas_call(
        _tc_c_kern,
        grid=grid,
        in_specs=[
            pl.BlockSpec((NC, BR, HID), lambda i: (0, i, 0)),
            pl.BlockSpec((BR, HID), lambda i: (i, 0)),
            pl.BlockSpec((BR, 1), lambda i: (i, 0)),
            pl.BlockSpec((BR, 1), lambda i: (i, 0)),
            pl.BlockSpec((HID, D), lambda i: (0, 0)),
            pl.BlockSpec((1, D), lambda i: (0, 0)),
        ],
        out_specs=pl.BlockSpec((BR, D), lambda i: (i, 0)),
        out_shape=jax.ShapeDtypeStruct((N, D), jnp.float32),
    )(zp, h, disv, dinvv, W2, b2)


def kernel(x, edge_index, edge_attr, W1, b1, W2, b2):
    src = edge_index[0].astype(jnp.int32)
    dst = edge_index[1].astype(jnp.int32)
    pad = EPAD - E
    spread = jnp.arange(pad, dtype=jnp.int32) % N
    src = jnp.concatenate([src, spread])
    dst = jnp.concatenate([dst, spread])
    ew = jnp.concatenate([edge_attr, jnp.zeros((pad,), jnp.float32)])
    srcc = src.reshape(TOTCH, CH)
    dstc = dst.reshape(TOTCH, CH)
    ewc = ew.reshape(TOTCH, CH)
    dst3 = dst.reshape(NW, NCHUNK, CH)
    ew3 = ew.reshape(NW, NCHUNK, CH)

    deg_parts = _sc_deg(dst3, ew3)
    deg = deg_parts[0, :N] + deg_parts[1, :N] + 1.0
    disv = lax.rsqrt(deg)[:, None]
    dinvv = (1.0 / deg)[:, None]

    xw1, y1 = _tc_a(x, W1, disv)
    z1p = _sc_agg(y1, srcc, dstc, ewc)
    h, y2 = _tc_b(z1p[:, :N], xw1, disv, dinvv, b1[None, :])
    z2p = _sc_agg(y2, srcc, dstc, ewc)
    out = _tc_c(z2p[:, :N], h, disv, dinvv, W2, b2[None, :])
    return out


# speedup vs baseline: 2.8002x; 1.0466x over previous
import functools

import jax
import jax.numpy as jnp
from jax import lax
from jax.experimental import pallas as pl
from jax.experimental.pallas import tpu as pltpu
from jax.experimental.pallas import tpu_sc as plsc

N = 10000
E = 320000
D = 128
HID = 64

NC = 2
NS = 16
LANES = 16
NW = NC * NS

CH = 128
NCHUNK = 80
EPW = CH * NCHUNK
EPAD = EPW * NW
TOTCH = EPAD // CH
CS0 = 80
CS1 = 80
CSF = max(CS0, CS1)
NPAD = 10240
RPW = NPAD // NS
ZR = 32
NBUF = 5

_mesh = plsc.VectorSubcoreMesh(core_axis_name="c", subcore_axis_name="s")


def _sc_deg_body(dst_hbm, ew_hbm, out_hbm, dstv, eww, zbuf, acc, sem):
    c = lax.axis_index("c")
    s = lax.axis_index("s")
    w = c * NS + s

    pltpu.sync_copy(dst_hbm.at[w], dstv)
    pltpu.sync_copy(ew_hbm.at[w], eww)

    def zf(i, _):
        zbuf[pl.ds(i * LANES, LANES)] = jnp.zeros((LANES,), jnp.float32)
        return 0
    lax.fori_loop(0, RPW // LANES, zf, 0)
    pltpu.sync_copy(zbuf, acc.at[pl.ds(s * RPW, RPW)])
    plsc.subcore_barrier()

    def fire(g, _):
        pltpu.async_copy(eww.at[g], acc.at[dstv.at[g]], sem, add=True)
        return 0
    lax.fori_loop(0, NCHUNK, fire, 0)

    def drain(g, _):
        pltpu.make_async_copy(eww.at[g], acc.at[dstv.at[g]], sem).wait()
        return 0
    lax.fori_loop(0, NCHUNK, drain, 0)

    plsc.subcore_barrier()
    pltpu.sync_copy(acc.at[pl.ds(s * RPW, RPW)], out_hbm.at[c, pl.ds(s * RPW, RPW)])


@functools.partial(
    pl.kernel,
    out_type=jax.ShapeDtypeStruct((NC, NPAD), jnp.float32),
    mesh=_mesh,
    scratch_types=[
        pltpu.VMEM((NCHUNK, CH), jnp.int32),
        pltpu.VMEM((NCHUNK, CH), jnp.float32),
        pltpu.VMEM((RPW,), jnp.float32),
        pltpu.VMEM_SHARED((NPAD,), jnp.float32),
        pltpu.SemaphoreType.DMA,
    ],
    compiler_params=pltpu.CompilerParams(use_tc_tiling_on_sc=False),
)
def _sc_deg(dst_hbm, ew_hbm, out_hbm, dstv, eww, zbuf, acc, sem):
    _sc_deg_body(dst_hbm, ew_hbm, out_hbm, dstv, eww, zbuf, acc, sem)


def _sc_agg_body(y_hbm, src_hbm, dst_hbm, ew_hbm, out_hbm,
                 srcv, dstv, eww, rows, zbuf, acc, gsem, ssem):
    c = lax.axis_index("c")
    s = lax.axis_index("s")

    cnt = jnp.where(c == 0, CS0, CS1)
    start = c * (NS * CS0) + s * cnt
    pltpu.sync_copy(src_hbm.at[pl.ds(start, CSF)], srcv)
    pltpu.sync_copy(dst_hbm.at[pl.ds(start, CSF)], dstv)
    pltpu.sync_copy(ew_hbm.at[pl.ds(start, CSF)], eww)

    def zf(i, _):
        for q in range(HID // LANES):
            zbuf[i, pl.ds(q * LANES, LANES)] = jnp.zeros((LANES,), jnp.float32)
        return 0
    lax.fori_loop(0, ZR, zf, 0)
    for r in range(RPW // ZR):
        pltpu.sync_copy(zbuf, acc.at[pl.ds(s * RPW + r * ZR, ZR)])
    plsc.subcore_barrier()

    def g_start(g, b):
        pltpu.async_copy(y_hbm.at[srcv.at[g]], rows.at[b], gsem)

    def g_wait(g, b):
        pltpu.make_async_copy(y_hbm.at[srcv.at[g]], rows.at[b], gsem).wait()

    def s_start(g, b):
        pltpu.async_copy(rows.at[b], acc.at[dstv.at[g]], ssem, add=True)

    def s_wait(g, b):
        pltpu.make_async_copy(rows.at[b], acc.at[dstv.at[g]], ssem).wait()

    def scale(g, b):
        @plsc.parallel_loop(0, CH // LANES, unroll=2)
        def sc16(j):
            ev = eww[g, pl.ds(j * LANES, LANES)]
            for k in range(LANES):
                nv = jnp.full((LANES,), ev[k], jnp.float32)
                e = j * LANES + k
                vals = [rows[b, e, pl.ds(q * LANES, LANES)] * nv
                        for q in range(HID // LANES)]
                for q in range(HID // LANES):
                    rows[b, e, pl.ds(q * LANES, LANES)] = vals[q]

    for b in range(NBUF):
        g_start(b, b)

    def outer(t, _):
        base = t * NBUF
        for b in range(NBUF):
            g = base + b

            @pl.when(t > 0)
            def _():
                s_wait(g - NBUF, b)
            g_wait(g, b)
            scale(g, b)
            s_start(g, b)

            @pl.when(g + NBUF < cnt)
            def _():
                g_start(g + NBUF, b)
        return 0
    lax.fori_loop(0, cnt // NBUF, outer, 0)
    for b in range(NBUF):
        s_wait(cnt - NBUF + b, b)

    plsc.subcore_barrier()
    pltpu.sync_copy(acc.at[pl.ds(s * RPW, RPW)],
                    out_hbm.at[c, pl.ds(s * RPW, RPW)])


@functools.partial(
    pl.kernel,
    out_type=jax.ShapeDtypeStruct((NC, NPAD, HID), jnp.float32),
    mesh=_mesh,
    scratch_types=[
        pltpu.VMEM((CSF, CH), jnp.int32),
        pltpu.VMEM((CSF, CH), jnp.int32),
        pltpu.VMEM((CSF, CH), jnp.float32),
        pltpu.VMEM((NBUF, CH, HID), jnp.float32),
        pltpu.VMEM((ZR, HID), jnp.float32),
        pltpu.VMEM_SHARED((NPAD, HID), jnp.float32),
        pltpu.SemaphoreType.DMA,
        pltpu.SemaphoreType.DMA,
    ],
    compiler_params=pltpu.CompilerParams(use_tc_tiling_on_sc=False),
)
def _sc_agg(y_hbm, src_hbm, dst_hbm, ew_hbm, out_hbm,
            srcv, dstv, eww, rows, zbuf, acc, gsem, ssem):
    _sc_agg_body(y_hbm, src_hbm, dst_hbm, ew_hbm, out_hbm,
                 srcv, dstv, eww, rows, zbuf, acc, gsem, ssem)


BR = 2000


def _tc_a_kern(x_ref, w_ref, dis_ref, xw_ref, y_ref):
    xw = jnp.dot(x_ref[...], w_ref[...], preferred_element_type=jnp.float32)
    xw_ref[...] = xw
    y_ref[...] = xw * dis_ref[...]


def _tc_a(x, W1, disv):
    grid = (N // BR,)
    return pl.pallas_call(
        _tc_a_kern,
        grid=grid,
        in_specs=[
            pl.BlockSpec((BR, D), lambda i: (i, 0)),
            pl.BlockSpec((D, HID), lambda i: (0, 0)),
            pl.BlockSpec((BR, 1), lambda i: (i, 0)),
        ],
        out_specs=[
            pl.BlockSpec((BR, HID), lambda i: (i, 0)),
            pl.BlockSpec((BR, HID), lambda i: (i, 0)),
        ],
        out_shape=[
            jax.ShapeDtypeStruct((N, HID), jnp.float32),
            jax.ShapeDtypeStruct((N, HID), jnp.float32),
        ],
    )(x, W1, disv)


def _tc_b_kern(zp_ref, xw_ref, dis_ref, dinv_ref, b_ref, h_ref, y2_ref):
    z = zp_ref[0] + zp_ref[1]
    h = jnp.maximum(z * dis_ref[...] + xw_ref[...] * dinv_ref[...] + b_ref[...],
                    0.0)
    h_ref[...] = h
    y2_ref[...] = h * dis_ref[...]


def _tc_b(zp, xw1, disv, dinvv, b1):
    grid = (N // BR,)
    return pl.pallas_call(
        _tc_b_kern,
        grid=grid,
        in_specs=[
            pl.BlockSpec((NC, BR, HID), lambda i: (0, i, 0)),
            pl.BlockSpec((BR, HID), lambda i: (i, 0)),
            pl.BlockSpec((BR, 1), lambda i: (i, 0)),
            pl.BlockSpec((BR, 1), lambda i: (i, 0)),
            pl.BlockSpec((1, HID), lambda i: (0, 0)),
        ],
        out_specs=[
            pl.BlockSpec((BR, HID), lambda i: (i, 0)),
            pl.BlockSpec((BR, HID), lambda i: (i, 0)),
        ],
        out_shape=[
            jax.ShapeDtypeStruct((N, HID), jnp.float32),
            jax.ShapeDtypeStruct((N, HID), jnp.float32),
        ],
    )(zp, xw1, disv, dinvv, b1)


def _tc_c_kern(zp_ref, h_ref, dis_ref, dinv_ref, w_ref, b_ref, o_ref):
    g = (zp_ref[0] + zp_ref[1]) * dis_ref[...] + h_ref[...] * dinv_ref[...]
    o_ref[...] = (jnp.dot(g, w_ref[...], preferred_element_type=jnp.float32)
                  + b_ref[...])


def _tc_c(zp, h, disv, dinvv, W2, b2):
    grid = (N // BR,)
    return pl.pallas_call(
        _tc_c_kern,
        grid=grid,
        in_specs=[
            pl.BlockSpec((NC, BR, HID), lambda i: (0, i, 0)),
            pl.BlockSpec((BR, HID), lambda i: (i, 0)),
            pl.BlockSpec((BR, 1), lambda i: (i, 0)),
            pl.BlockSpec((BR, 1), lambda i: (i, 0)),
            pl.BlockSpec((HID, D), lambda i: (0, 0)),
            pl.BlockSpec((1, D), lambda i: (0, 0)),
        ],
        out_specs=pl.BlockSpec((BR, D), lambda i: (i, 0)),
        out_shape=jax.ShapeDtypeStruct((N, D), jnp.float32),
    )(zp, h, disv, dinvv, W2, b2)


def kernel(x, edge_index, edge_attr, W1, b1, W2, b2):
    src = edge_index[0].astype(jnp.int32)
    dst = edge_index[1].astype(jnp.int32)
    pad = EPAD - E
    spread = jnp.arange(pad, dtype=jnp.int32) % N
    src = jnp.concatenate([src, spread])
    dst = jnp.concatenate([dst, spread])
    ew = jnp.concatenate([edge_attr, jnp.zeros((pad,), jnp.float32)])
    srcc = src.reshape(TOTCH, CH)
    dstc = dst.reshape(TOTCH, CH)
    ewc = ew.reshape(TOTCH, CH)
    dst3 = dst.reshape(NW, NCHUNK, CH)
    ew3 = ew.reshape(NW, NCHUNK, CH)

    deg_parts = _sc_deg(dst3, ew3)
    deg = deg_parts[0, :N] + deg_parts[1, :N] + 1.0
    disv = lax.rsqrt(deg)[:, None]
    dinvv = (1.0 / deg)[:, None]

    xw1, y1 = _tc_a(x, W1, disv)
    z1p = _sc_agg(y1, srcc, dstc, ewc)
    h, y2 = _tc_b(z1p[:, :N], xw1, disv, dinvv, b1[None, :])
    z2p = _sc_agg(y2, srcc, dstc, ewc)
    out = _tc_c(z2p[:, :N], h, disv, dinvv, W2, b2[None, :])
    return out
